# speedup vs baseline: 9.3350x; 9.3350x over previous
import functools

import jax
import jax.numpy as jnp
from jax.experimental import pallas as pl
from jax.experimental.pallas import tpu as pltpu

B, N = 4, 2048
D2D, D3D, DF, K, GH = 128, 128, 256, 8, 128
GATE_TEMP = 1.2
TOK = B * N
TBLK = 512


def _ln(x, g, b):
    m = x.mean(axis=-1, keepdims=True)
    v = ((x - m) ** 2).mean(axis=-1, keepdims=True)
    return (x - m) * jax.lax.rsqrt(v + 1e-5) * g + b


def _gelu(x):
    return x * 0.5 * (1.0 + jax.lax.erf(x * 0.7071067811865476))


def _dense_body(h2d_ref, h3d_ref, W_in_ref, b_in_ref, g_ln_g_ref, g_ln_b_ref,
                g_W1_ref, g_b1_ref, g_W2_ref, g_b2_ref, s_ln_g_ref, s_ln_b_ref,
                s_W_ref, s_b_ref, e_ln_g_ref, e_ln_b_ref, e_W1_ref, e_b1_ref,
                e_W2_ref, e_b2_ref, o_W_ref, o_b_ref, out_ref):
    f32 = jnp.float32
    x = (jnp.dot(h2d_ref[...], W_in_ref[:D2D, :], preferred_element_type=f32)
         + jnp.dot(h3d_ref[...], W_in_ref[D2D:, :], preferred_element_type=f32)
         + b_in_ref[...])
    base = jnp.dot(_gelu(_ln(x, s_ln_g_ref[...], s_ln_b_ref[...])), s_W_ref[...],
                   preferred_element_type=f32) + s_b_ref[...]
    g = _ln(base, g_ln_g_ref[...], g_ln_b_ref[...])
    g = _gelu(jnp.dot(g, g_W1_ref[...], preferred_element_type=f32) + g_b1_ref[...])
    logits = jnp.dot(g, g_W2_ref[...], preferred_element_type=f32) + g_b2_ref[...]
    scores = logits / GATE_TEMP
    kio = jax.lax.broadcasted_iota(jnp.int32, scores.shape, 1)
    v1 = jnp.max(scores, axis=-1, keepdims=True)
    i1 = jnp.min(jnp.where(scores == v1, kio, K), axis=-1, keepdims=True)
    oh1 = kio == i1
    masked = jnp.where(oh1, -jnp.inf, scores)
    v2 = jnp.max(masked, axis=-1, keepdims=True)
    i2 = jnp.min(jnp.where(masked == v2, kio, K), axis=-1, keepdims=True)
    oh2 = kio == i2
    e2 = jnp.exp(v2 - v1)
    p1 = 1.0 / (1.0 + e2)
    p2 = 1.0 - p1
    probs = jnp.where(oh1, p1, 0.0) + jnp.where(oh2, p2, 0.0)
    acc = jnp.zeros_like(base)
    for k in range(K):
        h = _ln(base, e_ln_g_ref[k, :], e_ln_b_ref[k, :])
        h = _gelu(jnp.dot(h, e_W1_ref[k], preferred_element_type=f32) + e_b1_ref[k, :])
        yk = jnp.dot(h, e_W2_ref[k], preferred_element_type=f32) + e_b2_ref[k, :]
        acc = acc + probs[:, k:k + 1] * yk
    out_ref[...] = jnp.dot(acc + base, o_W_ref[...], preferred_element_type=f32) + o_b_ref[...]


def kernel(h2d, h3d, W_in, b_in, g_ln_g, g_ln_b, g_W1, g_b1, g_W2, g_b2,
           s_ln_g, s_ln_b, s_W, s_b, e_ln_g, e_ln_b, e_W1, e_b1, e_W2, e_b2,
           o_W, o_b):
    h2 = h2d.reshape(TOK, D2D)
    h3 = h3d.reshape(TOK, D3D)
    nblk = TOK // TBLK

    def tok_spec(d):
        return pl.BlockSpec((TBLK, d), lambda i: (i, 0))

    def full_spec(arr):
        nd = arr.ndim
        return pl.BlockSpec(arr.shape, lambda i: (0,) * nd)

    full = [W_in, b_in, g_ln_g, g_ln_b, g_W1, g_b1, g_W2, g_b2,
            s_ln_g, s_ln_b, s_W, s_b, e_ln_g, e_ln_b, e_W1, e_b1,
            e_W2, e_b2, o_W, o_b]
    out = pl.pallas_call(
        _dense_body,
        grid=(nblk,),
        in_specs=[tok_spec(D2D), tok_spec(D3D)] + [full_spec(a) for a in full],
        out_specs=pl.BlockSpec((TBLK, DF), lambda i: (i, 0)),
        out_shape=jax.ShapeDtypeStruct((TOK, DF), jnp.float32),
    )(h2, h3, *full)
    return out.reshape(B, N, DF)
